# baseline (speedup 1.0000x reference)
import jax
import jax.numpy as jnp
from jax.experimental import pallas as pl
from jax.experimental.pallas import tpu as pltpu

E = 16
TILE = 512
D = 2048
NBUF = 6


def _copy_in(x_hbm, xbuf, insem, g, slot):
    return pltpu.make_async_copy(
        x_hbm.at[pl.ds(g * TILE, TILE), :], xbuf.at[slot], insem.at[slot])


def _router_body(x_hbm, w_ref, wout_hbm, tw_hbm, te_hbm,
                 xbuf, wbuf, twbuf, tebuf, insem, outsem):
    g = pl.program_id(0)
    nch = pl.num_programs(0)

    @pl.when(g == 0)
    def _prime():
        for s in range(NBUF - 1):
            _copy_in(x_hbm, xbuf, insem, s, s).start()

    nxt = g + NBUF - 1

    @pl.when(nxt < nch)
    def _prefetch():
        _copy_in(x_hbm, xbuf, insem, nxt, jax.lax.rem(nxt, NBUF)).start()

    slot = jax.lax.rem(g, NBUF)
    _copy_in(x_hbm, xbuf, insem, g, slot).wait()

    x = xbuf[slot]
    w = w_ref[...]
    lt = jax.lax.dot_general(w, x, (((0,), (1,)), ((), ())),
                             preferred_element_type=jnp.float32)
    m = jnp.max(lt, axis=0, keepdims=True)
    ex = jnp.exp(lt - m)
    s = jnp.sum(ex, axis=0, keepdims=True)
    rows = pl.ds(g * TILE, TILE)
    wbuf[rows, :] = (ex / s).T

    row = jax.lax.broadcasted_iota(jnp.int32, lt.shape, 0)
    i1 = jnp.min(jnp.where(lt == m, row, E), axis=0, keepdims=True)
    masked = jnp.where(row == i1, -jnp.inf, lt)
    l2 = jnp.max(masked, axis=0, keepdims=True)
    i2 = jnp.min(jnp.where(masked == l2, row, E), axis=0, keepdims=True)
    e2 = jnp.exp(l2 - m)
    r = 1.0 / (1.0 + e2)
    twbuf[rows, :] = jnp.concatenate([r, e2 * r], axis=0).T
    tebuf[rows, :] = jnp.concatenate([i1, i2], axis=0).T

    @pl.when(g == nch - 1)
    def _flush():
        c0 = pltpu.make_async_copy(wbuf, wout_hbm, outsem.at[0])
        c1 = pltpu.make_async_copy(twbuf, tw_hbm, outsem.at[1])
        c2 = pltpu.make_async_copy(tebuf, te_hbm, outsem.at[2])
        c0.start()
        c1.start()
        c2.start()
        c0.wait()
        c1.wait()
        c2.wait()


def kernel(x, W):
    B, S, _ = x.shape
    N = B * S
    x2 = x.reshape(N, D)
    grid = (N // TILE,)
    weights, topw, tope = pl.pallas_call(
        _router_body,
        grid=grid,
        in_specs=[
            pl.BlockSpec(memory_space=pl.ANY),
            pl.BlockSpec((D, E), lambda i: (0, 0)),
        ],
        out_specs=[
            pl.BlockSpec(memory_space=pl.ANY),
            pl.BlockSpec(memory_space=pl.ANY),
            pl.BlockSpec(memory_space=pl.ANY),
        ],
        out_shape=[
            jax.ShapeDtypeStruct((N, E), jnp.float32),
            jax.ShapeDtypeStruct((N, 2), jnp.float32),
            jax.ShapeDtypeStruct((N, 2), jnp.int32),
        ],
        scratch_shapes=[
            pltpu.VMEM((NBUF, TILE, D), jnp.float32),
            pltpu.VMEM((N, E), jnp.float32),
            pltpu.VMEM((N, 2), jnp.float32),
            pltpu.VMEM((N, 2), jnp.int32),
            pltpu.SemaphoreType.DMA((NBUF,)),
            pltpu.SemaphoreType.DMA((3,)),
        ],
    )(x2, W)
    return (
        weights.reshape(B, S, E),
        topw.reshape(B, S, 2),
        tope.reshape(B, S, 2),
    )
